# baseline (speedup 1.0000x reference)
import functools

import jax
import jax.numpy as jnp
from jax import lax
from jax.experimental import pallas as pl
from jax.experimental.pallas import tpu as pltpu
from jax.experimental.pallas import tpu_sc as plsc

NUM_WORDS = 1000000
DIM_EMBED = 32
NUM_CLASSES = 2
BATCH = 4096
SEQ = 200

NW = 32
CHUNK = 100
CHUNKS_PER_ROW = SEQ // CHUNK
ROWS_PER_W = BATCH // NW
CHUNKS_PER_W = ROWS_PER_W * CHUNKS_PER_ROW
HALF = 16
NBUF = 8
STREAM = 112
PROJ_BLK = 8192


def _tc_project(table, w):
    def body(w_ref, t_ref, o0_ref, o1_ref):
        t = t_ref[:]
        o0_ref[:] = jnp.dot(t, w_ref[0], preferred_element_type=jnp.float32)
        o1_ref[:] = jnp.dot(t, w_ref[1], preferred_element_type=jnp.float32)

    return pl.pallas_call(
        body,
        grid=((NUM_WORDS + PROJ_BLK - 1) // PROJ_BLK,),
        in_specs=[
            pl.BlockSpec((NUM_CLASSES, DIM_EMBED), lambda j: (0, 0)),
            pl.BlockSpec((PROJ_BLK, DIM_EMBED), lambda j: (j, 0)),
        ],
        out_specs=[
            pl.BlockSpec((PROJ_BLK,), lambda j: (j,)),
            pl.BlockSpec((PROJ_BLK,), lambda j: (j,)),
        ],
        out_shape=[
            jax.ShapeDtypeStruct((NUM_WORDS,), jnp.float32),
            jax.ShapeDtypeStruct((NUM_WORDS,), jnp.float32),
        ],
    )(w, table)


def _sc_gather_sums(x_pad, p0, p1):
    mesh = plsc.VectorSubcoreMesh(core_axis_name="c", subcore_axis_name="s")

    @functools.partial(
        pl.kernel,
        out_type=jax.ShapeDtypeStruct((NW * CHUNKS_PER_W * 2 * HALF,),
                                      jnp.float32),
        mesh=mesh,
        scratch_types=[
            pltpu.VMEM((CHUNKS_PER_W * 128,), jnp.int32),
            pltpu.VMEM((NBUF, 2, STREAM), jnp.float32),
            pltpu.VMEM((CHUNKS_PER_W * 2 * HALF,), jnp.float32),
            pltpu.SemaphoreType.DMA((NBUF,)),
        ],
        compiler_params=pltpu.CompilerParams(use_tc_tiling_on_sc=False),
    )
    def k(x_hbm, p0_hbm, p1_hbm, out_hbm, idx_v, vals_v, sums_v, sem):
        wid = lax.axis_index("s") * 2 + lax.axis_index("c")
        base = wid * CHUNKS_PER_W * 128
        pltpu.sync_copy(x_hbm.at[pl.ds(base, CHUNKS_PER_W * 128)], idx_v)

        tail_mask = lax.iota(jnp.int32, HALF) < (CHUNK % HALF)

        def gather(t, b):
            isl = idx_v.at[pl.ds(t * 128, STREAM)]
            pltpu.make_async_copy(
                p0_hbm.at[isl], vals_v.at[b, 0], sem.at[b]).start()
            pltpu.make_async_copy(
                p1_hbm.at[isl], vals_v.at[b, 1], sem.at[b]).start()

        def drain(t, b):
            isl = idx_v.at[pl.ds(t * 128, STREAM)]
            pltpu.make_async_copy(
                p0_hbm.at[isl], vals_v.at[b, 0], sem.at[b]).wait()
            pltpu.make_async_copy(
                p1_hbm.at[isl], vals_v.at[b, 1], sem.at[b]).wait()

        for b in range(NBUF):
            gather(b, b)

        def group_body(g, _):
            t0 = g * NBUF
            for b in range(NBUF):
                t = t0 + b
                drain(t, b)
                acc = [jnp.zeros((HALF,), jnp.float32) for _ in range(4)]
                for c in range(2):
                    for u in range(STREAM // HALF):
                        v = vals_v[b, c, pl.ds(u * HALF, HALF)]
                        if (u + 1) * HALF > CHUNK:
                            v = jnp.where(tail_mask, v, 0.0)
                        acc[u % 2 + 2 * c] = acc[u % 2 + 2 * c] + v

                @pl.when(g < CHUNKS_PER_W // NBUF - 1)
                def _():
                    gather(t + NBUF, b)

                sums_v[pl.ds(t * 2 * HALF, HALF)] = acc[0] + acc[1]
                sums_v[pl.ds(t * 2 * HALF + HALF, HALF)] = acc[2] + acc[3]
            return 0

        lax.fori_loop(0, CHUNKS_PER_W // NBUF, group_body, 0)
        pltpu.sync_copy(
            sums_v,
            out_hbm.at[pl.ds(wid * CHUNKS_PER_W * 2 * HALF,
                             CHUNKS_PER_W * 2 * HALF)])

    return k(x_pad, p0, p1)


def _tc_fold(sums, bias):
    def body(s_ref, b_ref, o_ref):
        s = s_ref[:]
        c0 = jnp.sum(s[:, :HALF], axis=1)
        c1 = jnp.sum(s[:, HALF:], axis=1)
        c = jnp.stack([c0, c1], axis=1)
        c = jnp.reshape(c, (BATCH, CHUNKS_PER_ROW, NUM_CLASSES))
        o_ref[:] = jnp.sum(c, axis=1) * (1.0 / SEQ) + b_ref[:]

    return pl.pallas_call(
        body,
        out_shape=jax.ShapeDtypeStruct((BATCH, NUM_CLASSES), jnp.float32),
    )(sums, bias)


def kernel(x, embedding_table, fc_weight, fc_bias):
    p0, p1 = _tc_project(embedding_table, fc_weight)
    x_flat = jnp.reshape(x.astype(jnp.int32), (-1, CHUNK))
    x_pad = jnp.reshape(jnp.pad(x_flat, ((0, 0), (0, 128 - CHUNK))), (-1,))
    sums = _sc_gather_sums(x_pad, p0, p1)
    sums2 = jnp.reshape(sums, (BATCH * CHUNKS_PER_ROW, 2 * HALF))
    return _tc_fold(sums2, jnp.reshape(fc_bias, (1, NUM_CLASSES)))


# speedup vs baseline: 1.2588x; 1.2588x over previous
import functools

import jax
import jax.numpy as jnp
from jax import lax
from jax.experimental import pallas as pl
from jax.experimental.pallas import tpu as pltpu
from jax.experimental.pallas import tpu_sc as plsc

NUM_WORDS = 1000000
DIM_EMBED = 32
NUM_CLASSES = 2
BATCH = 4096
SEQ = 200

NW = 32
CHUNK = 100
CHUNKS_PER_ROW = SEQ // CHUNK
ROWS_PER_W = BATCH // NW
CHUNKS_PER_W = ROWS_PER_W * CHUNKS_PER_ROW
HALF = 16
NBUF = 8
STREAM = 112
VREGS = STREAM // HALF
PROJ_BLK = 10000


def _tc_project(table4, w0, w1):
    def body(w0_ref, w1_ref, t_ref, o0_ref, o1_ref):
        t = t_ref[:]
        o0_ref[:] = jnp.dot(t, w0_ref[:], preferred_element_type=jnp.float32)
        o1_ref[:] = jnp.dot(t, w1_ref[:], preferred_element_type=jnp.float32)

    n4 = NUM_WORDS // 4
    return pl.pallas_call(
        body,
        grid=(n4 // PROJ_BLK,),
        in_specs=[
            pl.BlockSpec((128, 4), lambda j: (0, 0)),
            pl.BlockSpec((128, 4), lambda j: (0, 0)),
            pl.BlockSpec((PROJ_BLK, 128), lambda j: (j, 0)),
        ],
        out_specs=[
            pl.BlockSpec((PROJ_BLK, 4), lambda j: (j, 0)),
            pl.BlockSpec((PROJ_BLK, 4), lambda j: (j, 0)),
        ],
        out_shape=[jax.ShapeDtypeStruct((n4, 4), jnp.float32)] * 2,
    )(w0, w1, table4)


def _sc_gather_sums(x_pad, p0, p1):
    mesh = plsc.VectorSubcoreMesh(core_axis_name="c", subcore_axis_name="s")
    nidx = CHUNKS_PER_W * 128
    osz = CHUNKS_PER_W * 2 * HALF

    @functools.partial(
        pl.kernel,
        out_type=jax.ShapeDtypeStruct((NW * osz,), jnp.float32),
        mesh=mesh,
        scratch_types=[
            pltpu.VMEM((nidx,), jnp.int32),
            pltpu.VMEM((NBUF, 2, STREAM), jnp.float32),
            pltpu.VMEM((osz,), jnp.float32),
            pltpu.SemaphoreType.DMA((NBUF, 2)),
        ],
        compiler_params=pltpu.CompilerParams(use_tc_tiling_on_sc=False),
    )
    def k(x_hbm, p0_hbm, p1_hbm, out_hbm, idx_v, vals_v, sums_v, sem):
        wid = lax.axis_index("s") * 2 + lax.axis_index("c")
        pltpu.sync_copy(x_hbm.at[pl.ds(wid * nidx, nidx)], idx_v)

        lane = lax.iota(jnp.int32, HALF)
        tail = lane < (CHUNK - (VREGS - 1) * HALF)

        def gather(t, b):
            isl = idx_v.at[pl.ds(t * 128, STREAM)]
            pltpu.make_async_copy(p0_hbm.at[isl], vals_v.at[b, 0],
                                  sem.at[b, 0]).start()
            pltpu.make_async_copy(p1_hbm.at[isl], vals_v.at[b, 1],
                                  sem.at[b, 1]).start()

        def drain(t, b):
            isl = idx_v.at[pl.ds(t * 128, STREAM)]
            pltpu.make_async_copy(p0_hbm.at[isl], vals_v.at[b, 0],
                                  sem.at[b, 0]).wait()
            pltpu.make_async_copy(p1_hbm.at[isl], vals_v.at[b, 1],
                                  sem.at[b, 1]).wait()

        for b in range(NBUF):
            gather(b, b)

        def group_body(g, _):
            t0 = g * NBUF
            for b in range(NBUF):
                t = t0 + b
                drain(t, b)
                for c in range(2):
                    v = [vals_v[b, c, pl.ds(u * HALF, HALF)]
                         for u in range(VREGS)]
                    vlast = jnp.where(tail, v[VREGS - 1], 0.0)
                    acc = (((v[0] + v[1]) + (v[2] + v[3]))
                           + ((v[4] + v[5]) + vlast))
                    sums_v[pl.ds(t * 2 * HALF + c * HALF, HALF)] = acc

                @pl.when(g < CHUNKS_PER_W // NBUF - 1)
                def _():
                    gather(t + NBUF, b)
            return 0

        lax.fori_loop(0, CHUNKS_PER_W // NBUF, group_body, 0)
        pltpu.sync_copy(sums_v, out_hbm.at[pl.ds(wid * osz, osz)])

    return k(x_pad, p0, p1)


def _tc_fold(s64, m64, bias):
    def body(s_ref, m_ref, b_ref, o_ref):
        o_ref[:] = jnp.dot(s_ref[:], m_ref[:],
                           preferred_element_type=jnp.float32) + b_ref[:]

    return pl.pallas_call(
        body,
        out_shape=jax.ShapeDtypeStruct((BATCH, NUM_CLASSES), jnp.float32),
    )(s64, m64, bias)


def kernel(x, embedding_table, fc_weight, fc_bias):
    table4 = jnp.reshape(embedding_table, (NUM_WORDS // 4, 128))
    eye4 = jnp.eye(4, dtype=jnp.float32)
    w0 = jnp.reshape(eye4[:, None, :] * fc_weight[0][None, :, None], (128, 4))
    w1 = jnp.reshape(eye4[:, None, :] * fc_weight[1][None, :, None], (128, 4))
    p0_, p1_ = _tc_project(table4, w0, w1)
    p0 = jnp.reshape(p0_, (NUM_WORDS,))
    p1 = jnp.reshape(p1_, (NUM_WORDS,))

    x_flat = jnp.reshape(x.astype(jnp.int32), (-1, CHUNK))
    x_pad = jnp.reshape(jnp.pad(x_flat, ((0, 0), (0, 128 - CHUNK))), (-1,))
    sums = _sc_gather_sums(x_pad, p0, p1)

    s64 = jnp.reshape(sums, (BATCH, 2 * NUM_CLASSES * HALF))
    col = jnp.arange(2 * NUM_CLASSES * HALF, dtype=jnp.int32)
    m64 = ((col[:, None] % 32) // HALF
           == jnp.arange(NUM_CLASSES)[None, :]).astype(jnp.float32) / SEQ
    return _tc_fold(s64, m64, jnp.reshape(fc_bias, (1, NUM_CLASSES)))


# speedup vs baseline: 2.9031x; 2.3063x over previous
import functools

import jax
import jax.numpy as jnp
from jax import lax
from jax.experimental import pallas as pl
from jax.experimental.pallas import tpu as pltpu
from jax.experimental.pallas import tpu_sc as plsc

NUM_WORDS = 1000000
DIM_EMBED = 32
NUM_CLASSES = 2
BATCH = 4096
SEQ = 200

NW = 32
CHUNK = 100
CHUNKS_PER_ROW = SEQ // CHUNK
ROWS_PER_W = BATCH // NW
CHUNKS_PER_W = ROWS_PER_W * CHUNKS_PER_ROW
HALF = 16
NBUF = 4


def _sc_gather_sums(x_pad, table):
    mesh = plsc.VectorSubcoreMesh(core_axis_name="c", subcore_axis_name="s")
    nidx = CHUNKS_PER_W * 128
    osz = CHUNKS_PER_W * DIM_EMBED

    @functools.partial(
        pl.kernel,
        out_type=jax.ShapeDtypeStruct((NW * osz,), jnp.float32),
        mesh=mesh,
        scratch_types=[
            pltpu.VMEM((nidx,), jnp.int32),
            pltpu.VMEM((NBUF, CHUNK, DIM_EMBED), jnp.float32),
            pltpu.VMEM((osz,), jnp.float32),
            pltpu.SemaphoreType.DMA((NBUF,)),
        ],
        compiler_params=pltpu.CompilerParams(use_tc_tiling_on_sc=False),
    )
    def k(x_hbm, t_hbm, out_hbm, idx_v, vals_v, sums_v, sem):
        wid = lax.axis_index("s") * 2 + lax.axis_index("c")
        pltpu.sync_copy(x_hbm.at[pl.ds(wid * nidx, nidx)], idx_v)

        def gather(t, b):
            isl = idx_v.at[pl.ds(t * 128, CHUNK)]
            pltpu.make_async_copy(t_hbm.at[isl], vals_v.at[b],
                                  sem.at[b]).start()

        def drain(t, b):
            isl = idx_v.at[pl.ds(t * 128, CHUNK)]
            pltpu.make_async_copy(t_hbm.at[isl], vals_v.at[b],
                                  sem.at[b]).wait()

        for b in range(NBUF):
            gather(b, b)

        def group_body(g, _):
            t0 = g * NBUF
            for b in range(NBUF):
                t = t0 + b
                drain(t, b)
                for h in range(2):
                    acc = [vals_v[b, i, pl.ds(h * HALF, HALF)]
                           for i in range(4)]
                    for i in range(4, CHUNK, 4):
                        for u in range(4):
                            acc[u] = acc[u] + vals_v[b, i + u,
                                                     pl.ds(h * HALF, HALF)]
                    sums_v[pl.ds(t * DIM_EMBED + h * HALF, HALF)] = (
                        (acc[0] + acc[1]) + (acc[2] + acc[3]))

                @pl.when(g < CHUNKS_PER_W // NBUF - 1)
                def _():
                    gather(t + NBUF, b)
            return 0

        lax.fori_loop(0, CHUNKS_PER_W // NBUF, group_body, 0)
        pltpu.sync_copy(sums_v, out_hbm.at[pl.ds(wid * osz, osz)])

    return k(x_pad, table)


def _tc_fold(s64, m64, bias):
    def body(s_ref, m_ref, b_ref, o_ref):
        o_ref[:] = jnp.dot(s_ref[:], m_ref[:],
                           preferred_element_type=jnp.float32) + b_ref[:]

    return pl.pallas_call(
        body,
        out_shape=jax.ShapeDtypeStruct((BATCH, NUM_CLASSES), jnp.float32),
    )(s64, m64, bias)


def kernel(x, embedding_table, fc_weight, fc_bias):
    x_flat = jnp.reshape(x.astype(jnp.int32), (-1, CHUNK))
    x_pad = jnp.reshape(jnp.pad(x_flat, ((0, 0), (0, 128 - CHUNK))), (-1,))
    sums = _sc_gather_sums(x_pad, embedding_table)

    s64 = jnp.reshape(sums, (BATCH, 2 * DIM_EMBED))
    m64 = jnp.concatenate([fc_weight.T, fc_weight.T], axis=0) / SEQ
    return _tc_fold(s64, m64, jnp.reshape(fc_bias, (1, NUM_CLASSES)))
